# baseline (speedup 1.0000x reference)
import functools

import jax
import jax.numpy as jnp
from jax import lax
from jax.experimental import pallas as pl
from jax.experimental.pallas import tpu as pltpu
from jax.experimental.pallas import tpu_sc as plsc

_EPS = 0.01
_NC = 2
_NS = 16
_NW = _NC * _NS
_L = 16


def _full(val):
    return jnp.full((_L,), val, jnp.int32)


def _routing_step(u_v, cand_v, noise_v, out_v, num_groups, k, nmax):
    iota16 = lax.iota(jnp.int32, _L) * k

    def body(g, carry):
        gbase = g * (k * _L)
        off = gbase + iota16
        s = []
        for j in range(k):
            dj = plsc.load_gather(cand_v, [off + _full(j)])
            dj = jnp.minimum(jnp.maximum(dj, 0), nmax)
            s.append(plsc.load_gather(u_v, [dj]))
        m = s[0]
        for j in range(1, k):
            m = jnp.maximum(m, s[j])
        e = [jnp.exp(x - m) for x in s]
        tot = e[0]
        for j in range(1, k):
            tot = tot + e[j]
        best = e[0] / tot + noise_v[pl.ds(gbase, _L)]
        bestj = _full(0)
        for j in range(1, k):
            p = e[j] / tot + noise_v[pl.ds(gbase + j * _L, _L)]
            gt = p > best
            bestj = jnp.where(gt, _full(j), bestj)
            best = jnp.where(gt, p, best)
        chosen = plsc.load_gather(cand_v, [off + bestj])
        chosen = jnp.minimum(jnp.maximum(chosen, 0), nmax)
        out_v[pl.ds(g * _L, _L)] = chosen
        return carry

    lax.fori_loop(0, num_groups, body, 0)


@functools.lru_cache(maxsize=4)
def _build_sc_router(n, npad, d, k):
    b = npad // _NW
    groups = b // _L
    bk = b * k
    tail = n - (_NW - 1) * b
    mesh = plsc.VectorSubcoreMesh(
        core_axis_name="c", subcore_axis_name="s",
        num_cores=_NC, num_subcores=_NS)

    @functools.partial(
        pl.kernel,
        out_type=[
            jax.ShapeDtypeStruct((n,), jnp.int32),
            jax.ShapeDtypeStruct((n,), jnp.int32),
            jax.ShapeDtypeStruct((n, 2 * d), jnp.float32),
        ],
        mesh=mesh,
        compiler_params=pltpu.CompilerParams(needs_layout_passes=False),
        scratch_types=[
            pltpu.VMEM((n,), jnp.float32),
            pltpu.VMEM((n,), jnp.float32),
            pltpu.VMEM((bk,), jnp.int32),
            pltpu.VMEM((bk,), jnp.float32),
            pltpu.VMEM((bk,), jnp.int32),
            pltpu.VMEM((b,), jnp.int32),
            pltpu.VMEM((b,), jnp.int32),
            pltpu.VMEM((b, d), jnp.float32),
            pltpu.VMEM((b, d), jnp.float32),
            pltpu.SemaphoreType.DMA,
            pltpu.SemaphoreType.DMA,
            pltpu.SemaphoreType.DMA,
            pltpu.SemaphoreType.DMA,
            pltpu.SemaphoreType.DMA,
            pltpu.SemaphoreType.DMA,
        ],
    )
    def sc_router(u8_hbm, adjrm_hbm, n0_hbm, n1_hbm, v_hbm,
                  w1_hbm, w2_hbm, emb_hbm,
                  u0_v, u1_v, cand_v, noise_v, sidx_v, w1_v, w2_v,
                  rows0_v, rows1_v,
                  s_u0, s_u1, s_cand, s_noise, s_e0, s_e1):
        wid = lax.axis_index("s") * _NC + lax.axis_index("c")
        base = wid * b

        d_u0 = pltpu.async_copy(u8_hbm.at[0], u0_v, s_u0)
        d_u1 = pltpu.async_copy(u8_hbm.at[1], u1_v, s_u1)
        d_n = pltpu.async_copy(n0_hbm.at[pl.ds(base * k, bk)], noise_v,
                               s_noise)

        @pl.when(wid < _NW - 1)
        def _load_full():
            pltpu.sync_copy(adjrm_hbm.at[pl.ds(base * k, bk)], cand_v)

        @pl.when(wid == _NW - 1)
        def _load_tail():
            pltpu.sync_copy(adjrm_hbm.at[pl.ds(base * k, tail * k)],
                            cand_v.at[pl.ds(0, tail * k)])

        d_u0.wait()
        d_n.wait()

        _routing_step(u0_v, cand_v, noise_v, w1_v, groups, k, n - 1)

        iota16 = lax.iota(jnp.int32, _L) * k

        def sidx_body(c, carry):
            wchunk = w1_v[pl.ds(c * _L, _L)] * k
            pos = c * (k * _L) + iota16
            for j in range(k):
                plsc.store_scatter(sidx_v, [pos + _full(j)],
                                   wchunk + _full(j))
            return carry

        lax.fori_loop(0, groups, sidx_body, 0)

        c_ds = []
        for c in range(groups):
            for h in range(2):
                off = c * (k * _L) + h * 128
                c_ds.append(pltpu.async_copy(
                    adjrm_hbm.at[sidx_v.at[pl.ds(off, 128)]],
                    cand_v.at[pl.ds(off, 128)], s_cand))
        d_n1 = pltpu.async_copy(n1_hbm.at[pl.ds(base * k, bk)], noise_v,
                                s_noise)
        e0_ds = []
        for c in range(groups):
            e0_ds.append(pltpu.async_copy(
                v_hbm.at[w1_v.at[pl.ds(c * _L, _L)]],
                rows0_v.at[pl.ds(c * _L, _L)], s_e0))
        for ds in c_ds:
            ds.wait()
        d_n1.wait()
        d_u1.wait()

        _routing_step(u1_v, cand_v, noise_v, w2_v, groups, k, n - 1)

        e1_ds = []
        for c in range(groups):
            e1_ds.append(pltpu.async_copy(
                v_hbm.at[w2_v.at[pl.ds(c * _L, _L)]],
                rows1_v.at[pl.ds(c * _L, _L)], s_e1))
        for ds in e0_ds:
            ds.wait()
        for ds in e1_ds:
            ds.wait()

        @pl.when(wid < _NW - 1)
        def _full_write():
            pltpu.sync_copy(w1_v, w1_hbm.at[pl.ds(base, b)])
            pltpu.sync_copy(w2_v, w2_hbm.at[pl.ds(base, b)])
            pltpu.sync_copy(rows0_v, emb_hbm.at[pl.ds(base, b), pl.ds(0, d)])
            pltpu.sync_copy(rows1_v, emb_hbm.at[pl.ds(base, b), pl.ds(d, d)])

        @pl.when(wid == _NW - 1)
        def _tail_write():
            pltpu.sync_copy(w1_v.at[pl.ds(0, tail)],
                            w1_hbm.at[pl.ds(base, tail)])
            pltpu.sync_copy(w2_v.at[pl.ds(0, tail)],
                            w2_hbm.at[pl.ds(base, tail)])
            pltpu.sync_copy(rows0_v.at[pl.ds(0, tail)],
                            emb_hbm.at[pl.ds(base, tail), pl.ds(0, d)])
            pltpu.sync_copy(rows1_v.at[pl.ds(0, tail)],
                            emb_hbm.at[pl.ds(base, tail), pl.ds(d, d)])

    return sc_router


def _u_body(w_ref, v_ref, o_ref):
    o_ref[...] = lax.dot_general(
        w_ref[...], v_ref[...], (((1,), (1,)), ((), ())),
        preferred_element_type=jnp.float32)


def kernel(v, adj, slices, W, b):
    n, d = v.shape
    k = adj.shape[1] // slices.shape[0]
    t = W.shape[0] // d
    assert t == 2 and k == _L
    npad = -(-n // (_NW * _L)) * (_NW * _L)

    w_pad = jnp.zeros((8, d), jnp.float32).at[:t].set(W[:, 0].reshape(t, d))
    noise = []
    for i in range(t):
        ni = _EPS * jax.random.normal(
            jax.random.fold_in(jax.random.key(1234), i), (n * k,),
            dtype=jnp.float32)
        nip = jnp.zeros((npad, k), jnp.float32).at[:n].set(ni.reshape(n, k))
        noise.append(nip.reshape(-1, _L, k).transpose(0, 2, 1).reshape(-1))

    u8 = pl.pallas_call(
        _u_body,
        out_shape=jax.ShapeDtypeStruct((8, n), jnp.float32),
    )(w_pad, v)

    w1, w2, walk_embeds = _build_sc_router(n, npad, d, k)(
        u8, adj[1].astype(jnp.int32), noise[0], noise[1], v)

    walks = jnp.stack([jnp.arange(n, dtype=jnp.int32), w1, w2], axis=1)
    return walks, walk_embeds


# speedup vs baseline: 1.0252x; 1.0252x over previous
import functools

import jax
import jax.numpy as jnp
from jax import lax
from jax.experimental import pallas as pl
from jax.experimental.pallas import tpu as pltpu
from jax.experimental.pallas import tpu_sc as plsc

_EPS = 0.01
_NC = 2
_NS = 16
_NW = _NC * _NS
_L = 16


def _full(val):
    return jnp.full((_L,), val, jnp.int32)


def _routing_step(u_v, cand_v, noise_v, out_v, num_groups, k):
    iota16 = lax.iota(jnp.int32, _L) * k

    @plsc.parallel_loop(0, num_groups, unroll=2)
    def body(g):
        gbase = g * (k * _L)
        off = gbase + iota16
        s = []
        for j in range(k):
            dj = plsc.load_gather(cand_v, [off + _full(j)])
            s.append(plsc.load_gather(u_v, [dj]))
        m = s[0]
        for j in range(1, k):
            m = jnp.maximum(m, s[j])
        e = [jnp.exp(x - m) for x in s]
        tot = e[0]
        for j in range(1, k):
            tot = tot + e[j]
        best = e[0] / tot + noise_v[pl.ds(gbase, _L)]
        bestj = _full(0)
        for j in range(1, k):
            p = e[j] / tot + noise_v[pl.ds(gbase + j * _L, _L)]
            gt = p > best
            bestj = jnp.where(gt, _full(j), bestj)
            best = jnp.where(gt, p, best)
        chosen = plsc.load_gather(cand_v, [off + bestj])
        out_v[pl.ds(g * _L, _L)] = chosen


@functools.lru_cache(maxsize=4)
def _build_sc_router(n, npad, d, k):
    b = npad // _NW
    groups = b // _L
    bk = b * k
    tail = n - (_NW - 1) * b
    mesh = plsc.VectorSubcoreMesh(
        core_axis_name="c", subcore_axis_name="s",
        num_cores=_NC, num_subcores=_NS)

    @functools.partial(
        pl.kernel,
        out_type=[
            jax.ShapeDtypeStruct((n,), jnp.int32),
            jax.ShapeDtypeStruct((n,), jnp.int32),
            jax.ShapeDtypeStruct((n, 2 * d), jnp.float32),
        ],
        mesh=mesh,
        compiler_params=pltpu.CompilerParams(needs_layout_passes=False),
        scratch_types=[
            pltpu.VMEM((n,), jnp.float32),
            pltpu.VMEM((n,), jnp.float32),
            pltpu.VMEM((bk,), jnp.int32),
            pltpu.VMEM((bk,), jnp.float32),
            pltpu.VMEM((bk,), jnp.int32),
            pltpu.VMEM((b,), jnp.int32),
            pltpu.VMEM((b,), jnp.int32),
            pltpu.VMEM((b, d), jnp.float32),
            pltpu.VMEM((b, d), jnp.float32),
            pltpu.SemaphoreType.DMA,
            pltpu.SemaphoreType.DMA,
            pltpu.SemaphoreType.DMA,
            pltpu.SemaphoreType.DMA,
            pltpu.SemaphoreType.DMA,
            pltpu.SemaphoreType.DMA,
        ],
    )
    def sc_router(u0_hbm, u1_hbm, adjrm_hbm, n0_hbm, n1_hbm, v_hbm,
                  w1_hbm, w2_hbm, emb_hbm,
                  u0_v, u1_v, cand_v, noise_v, sidx_v, w1_v, w2_v,
                  rows0_v, rows1_v,
                  s_u0, s_u1, s_cand, s_noise, s_e0, s_e1):
        wid = lax.axis_index("s") * _NC + lax.axis_index("c")
        base = wid * b

        d_u0 = pltpu.async_copy(u0_hbm, u0_v, s_u0)
        d_u1 = pltpu.async_copy(u1_hbm, u1_v, s_u1)
        d_c = pltpu.async_copy(adjrm_hbm.at[pl.ds(base * k, bk)], cand_v,
                               s_cand)
        d_n = pltpu.async_copy(n0_hbm.at[pl.ds(base * k, bk)], noise_v,
                               s_noise)
        d_u0.wait()
        d_c.wait()
        d_n.wait()

        _routing_step(u0_v, cand_v, noise_v, w1_v, groups, k)

        iota16 = lax.iota(jnp.int32, _L) * k

        @plsc.parallel_loop(0, groups, unroll=2)
        def sidx_body(c):
            wchunk = w1_v[pl.ds(c * _L, _L)] * k
            pos = c * (k * _L) + iota16
            for j in range(k):
                plsc.store_scatter(sidx_v, [pos + _full(j)],
                                   wchunk + _full(j))

        c_ds = []
        for c in range(groups):
            for h in range(2):
                off = c * (k * _L) + h * 128
                c_ds.append(pltpu.async_copy(
                    adjrm_hbm.at[sidx_v.at[pl.ds(off, 128)]],
                    cand_v.at[pl.ds(off, 128)], s_cand))
        d_n1 = pltpu.async_copy(n1_hbm.at[pl.ds(base * k, bk)], noise_v,
                                s_noise)
        e0_ds = []
        for c in range(groups):
            e0_ds.append(pltpu.async_copy(
                v_hbm.at[w1_v.at[pl.ds(c * _L, _L)]],
                rows0_v.at[pl.ds(c * _L, _L)], s_e0))
        for ds in c_ds:
            ds.wait()
        d_n1.wait()
        d_u1.wait()

        _routing_step(u1_v, cand_v, noise_v, w2_v, groups, k)

        e1_ds = []
        for c in range(groups):
            e1_ds.append(pltpu.async_copy(
                v_hbm.at[w2_v.at[pl.ds(c * _L, _L)]],
                rows1_v.at[pl.ds(c * _L, _L)], s_e1))
        for ds in e0_ds:
            ds.wait()
        for ds in e1_ds:
            ds.wait()

        @pl.when(wid < _NW - 1)
        def _full_write():
            pltpu.sync_copy(w1_v, w1_hbm.at[pl.ds(base, b)])
            pltpu.sync_copy(w2_v, w2_hbm.at[pl.ds(base, b)])
            pltpu.sync_copy(rows0_v, emb_hbm.at[pl.ds(base, b), pl.ds(0, d)])
            pltpu.sync_copy(rows1_v, emb_hbm.at[pl.ds(base, b), pl.ds(d, d)])

        @pl.when(wid == _NW - 1)
        def _tail_write():
            pltpu.sync_copy(w1_v.at[pl.ds(0, tail)],
                            w1_hbm.at[pl.ds(base, tail)])
            pltpu.sync_copy(w2_v.at[pl.ds(0, tail)],
                            w2_hbm.at[pl.ds(base, tail)])
            pltpu.sync_copy(rows0_v.at[pl.ds(0, tail)],
                            emb_hbm.at[pl.ds(base, tail), pl.ds(0, d)])
            pltpu.sync_copy(rows1_v.at[pl.ds(0, tail)],
                            emb_hbm.at[pl.ds(base, tail), pl.ds(d, d)])

    return sc_router


def _u_body(w_ref, v_ref, o_ref):
    o_ref[...] = lax.dot_general(
        w_ref[...], v_ref[...], (((1,), (1,)), ((), ())),
        preferred_element_type=jnp.float32)


def kernel(v, adj, slices, W, b):
    n, d = v.shape
    k = adj.shape[1] // slices.shape[0]
    t = W.shape[0] // d
    assert t == 2 and k == _L
    npad = -(-n // (_NW * _L)) * (_NW * _L)

    adj_rm = jnp.concatenate(
        [adj[1].astype(jnp.int32), jnp.zeros((npad - n) * k, jnp.int32)])
    w_pad = jnp.zeros((8, d), jnp.float32).at[:t].set(W[:, 0].reshape(t, d))
    noise = []
    for i in range(t):
        ni = _EPS * jax.random.normal(
            jax.random.fold_in(jax.random.key(1234), i), (n * k,),
            dtype=jnp.float32)
        nip = jnp.zeros((npad, k), jnp.float32).at[:n].set(ni.reshape(n, k))
        noise.append(nip.reshape(-1, _L, k).transpose(0, 2, 1).reshape(-1))

    u8 = pl.pallas_call(
        _u_body,
        out_shape=jax.ShapeDtypeStruct((8, n), jnp.float32),
    )(w_pad, v)

    w1, w2, walk_embeds = _build_sc_router(n, npad, d, k)(
        u8[0], u8[1], adj_rm, noise[0], noise[1], v)

    walks = jnp.stack([jnp.arange(n, dtype=jnp.int32), w1, w2], axis=1)
    return walks, walk_embeds


# speedup vs baseline: 1.0379x; 1.0125x over previous
import functools

import jax
import jax.numpy as jnp
from jax import lax
from jax.experimental import pallas as pl
from jax.experimental.pallas import tpu as pltpu
from jax.experimental.pallas import tpu_sc as plsc

_EPS = 0.01
_NC = 2
_NS = 16
_NW = _NC * _NS
_L = 16


def _full(val):
    return jnp.full((_L,), val, jnp.int32)


def _routing_step(u_v, cand_v, noise_v, out_v, num_groups, k):
    iota16 = lax.iota(jnp.int32, _L) * k

    @plsc.parallel_loop(0, num_groups, unroll=2)
    def body(g):
        gbase = g * (k * _L)
        off = gbase + iota16
        s = []
        for j in range(k):
            dj = plsc.load_gather(cand_v, [off + _full(j)])
            s.append(plsc.load_gather(u_v, [dj]))
        m = s[0]
        for j in range(1, k):
            m = jnp.maximum(m, s[j])
        e = [jnp.exp(x - m) for x in s]
        tot = e[0]
        for j in range(1, k):
            tot = tot + e[j]
        best = e[0] / tot + noise_v[pl.ds(gbase, _L)]
        bestj = _full(0)
        for j in range(1, k):
            p = e[j] / tot + noise_v[pl.ds(gbase + j * _L, _L)]
            gt = p > best
            bestj = jnp.where(gt, _full(j), bestj)
            best = jnp.where(gt, p, best)
        chosen = plsc.load_gather(cand_v, [off + bestj])
        out_v[pl.ds(g * _L, _L)] = chosen


@functools.lru_cache(maxsize=4)
def _build_sc_router(n, npad, d, k):
    b = npad // _NW
    groups = b // _L
    bk = b * k
    tail = n - (_NW - 1) * b
    mesh = plsc.VectorSubcoreMesh(
        core_axis_name="c", subcore_axis_name="s",
        num_cores=_NC, num_subcores=_NS)

    @functools.partial(
        pl.kernel,
        out_type=[
            jax.ShapeDtypeStruct((n,), jnp.int32),
            jax.ShapeDtypeStruct((n,), jnp.int32),
            jax.ShapeDtypeStruct((n, 2 * d), jnp.float32),
        ],
        mesh=mesh,
        compiler_params=pltpu.CompilerParams(
            needs_layout_passes=False, disable_bounds_checks=True,
            disable_semaphore_checks=True),
        scratch_types=[
            pltpu.VMEM((n,), jnp.float32),
            pltpu.VMEM((n,), jnp.float32),
            pltpu.VMEM((bk,), jnp.int32),
            pltpu.VMEM((bk,), jnp.float32),
            pltpu.VMEM((bk,), jnp.int32),
            pltpu.VMEM((b,), jnp.int32),
            pltpu.VMEM((b,), jnp.int32),
            pltpu.VMEM((b, d), jnp.float32),
            pltpu.VMEM((b, d), jnp.float32),
            pltpu.SemaphoreType.DMA,
            pltpu.SemaphoreType.DMA,
            pltpu.SemaphoreType.DMA,
            pltpu.SemaphoreType.DMA,
            pltpu.SemaphoreType.DMA,
            pltpu.SemaphoreType.DMA,
        ],
    )
    def sc_router(u0_hbm, u1_hbm, adjrm_hbm, n0_hbm, n1_hbm, v_hbm,
                  w1_hbm, w2_hbm, emb_hbm,
                  u0_v, u1_v, cand_v, noise_v, sidx_v, w1_v, w2_v,
                  rows0_v, rows1_v,
                  s_u0, s_u1, s_cand, s_noise, s_e0, s_e1):
        wid = lax.axis_index("s") * _NC + lax.axis_index("c")
        base = wid * b

        d_u0 = pltpu.async_copy(u0_hbm, u0_v, s_u0)
        d_u1 = pltpu.async_copy(u1_hbm, u1_v, s_u1)
        d_c = pltpu.async_copy(adjrm_hbm.at[pl.ds(base * k, bk)], cand_v,
                               s_cand)
        d_n = pltpu.async_copy(n0_hbm.at[pl.ds(base * k, bk)], noise_v,
                               s_noise)
        d_u0.wait()
        d_c.wait()
        d_n.wait()

        _routing_step(u0_v, cand_v, noise_v, w1_v, groups, k)

        iota16 = lax.iota(jnp.int32, _L) * k

        @plsc.parallel_loop(0, groups, unroll=2)
        def sidx_body(c):
            wchunk = w1_v[pl.ds(c * _L, _L)] * k
            pos = c * (k * _L) + iota16
            for j in range(k):
                plsc.store_scatter(sidx_v, [pos + _full(j)],
                                   wchunk + _full(j))

        c_ds = []
        for c in range(groups):
            for h in range(2):
                off = c * (k * _L) + h * 128
                c_ds.append(pltpu.async_copy(
                    adjrm_hbm.at[sidx_v.at[pl.ds(off, 128)]],
                    cand_v.at[pl.ds(off, 128)], s_cand))
        d_n1 = pltpu.async_copy(n1_hbm.at[pl.ds(base * k, bk)], noise_v,
                                s_noise)
        e0_ds = []
        for c in range(groups):
            e0_ds.append(pltpu.async_copy(
                v_hbm.at[w1_v.at[pl.ds(c * _L, _L)]],
                rows0_v.at[pl.ds(c * _L, _L)], s_e0))
        for ds in c_ds:
            ds.wait()
        d_n1.wait()
        d_u1.wait()

        _routing_step(u1_v, cand_v, noise_v, w2_v, groups, k)

        e1_ds = []
        for c in range(groups):
            e1_ds.append(pltpu.async_copy(
                v_hbm.at[w2_v.at[pl.ds(c * _L, _L)]],
                rows1_v.at[pl.ds(c * _L, _L)], s_e1))
        for ds in e0_ds:
            ds.wait()
        for ds in e1_ds:
            ds.wait()

        @pl.when(wid < _NW - 1)
        def _full_write():
            pltpu.sync_copy(w1_v, w1_hbm.at[pl.ds(base, b)])
            pltpu.sync_copy(w2_v, w2_hbm.at[pl.ds(base, b)])
            pltpu.sync_copy(rows0_v, emb_hbm.at[pl.ds(base, b), pl.ds(0, d)])
            pltpu.sync_copy(rows1_v, emb_hbm.at[pl.ds(base, b), pl.ds(d, d)])

        @pl.when(wid == _NW - 1)
        def _tail_write():
            pltpu.sync_copy(w1_v.at[pl.ds(0, tail)],
                            w1_hbm.at[pl.ds(base, tail)])
            pltpu.sync_copy(w2_v.at[pl.ds(0, tail)],
                            w2_hbm.at[pl.ds(base, tail)])
            pltpu.sync_copy(rows0_v.at[pl.ds(0, tail)],
                            emb_hbm.at[pl.ds(base, tail), pl.ds(0, d)])
            pltpu.sync_copy(rows1_v.at[pl.ds(0, tail)],
                            emb_hbm.at[pl.ds(base, tail), pl.ds(d, d)])

    return sc_router


def _u_body(w_ref, v_ref, o_ref):
    o_ref[...] = lax.dot_general(
        w_ref[...], v_ref[...], (((1,), (1,)), ((), ())),
        preferred_element_type=jnp.float32)


def kernel(v, adj, slices, W, b):
    n, d = v.shape
    k = adj.shape[1] // slices.shape[0]
    t = W.shape[0] // d
    assert t == 2 and k == _L
    npad = -(-n // (_NW * _L)) * (_NW * _L)

    adj_rm = jnp.concatenate(
        [adj[1].astype(jnp.int32), jnp.zeros((npad - n) * k, jnp.int32)])
    w_pad = jnp.zeros((8, d), jnp.float32).at[:t].set(W[:, 0].reshape(t, d))
    noise = []
    for i in range(t):
        ni = _EPS * jax.random.normal(
            jax.random.fold_in(jax.random.key(1234), i), (n * k,),
            dtype=jnp.float32)
        nip = jnp.zeros((npad, k), jnp.float32).at[:n].set(ni.reshape(n, k))
        noise.append(nip.reshape(-1, _L, k).transpose(0, 2, 1).reshape(-1))

    u8 = pl.pallas_call(
        _u_body,
        out_shape=jax.ShapeDtypeStruct((8, n), jnp.float32),
    )(w_pad, v)

    w1, w2, walk_embeds = _build_sc_router(n, npad, d, k)(
        u8[0], u8[1], adj_rm, noise[0], noise[1], v)

    walks = jnp.stack([jnp.arange(n, dtype=jnp.int32), w1, w2], axis=1)
    return walks, walk_embeds


# speedup vs baseline: 1.0518x; 1.0134x over previous
import functools

import jax
import jax.numpy as jnp
from jax import lax
from jax.experimental import pallas as pl
from jax.experimental.pallas import tpu as pltpu
from jax.experimental.pallas import tpu_sc as plsc

_EPS = 0.01
_NC = 2
_NS = 16
_NW = _NC * _NS
_L = 16


def _full(val):
    return jnp.full((_L,), val, jnp.int32)


def _routing_step(u_v, cand_v, noise_v, out_v, num_groups, k):
    iota16 = lax.iota(jnp.int32, _L) * k

    @plsc.parallel_loop(0, num_groups, unroll=2)
    def body(g):
        gbase = g * (k * _L)
        off = gbase + iota16
        s = []
        for j in range(k):
            dj = plsc.load_gather(cand_v, [off + _full(j)])
            s.append(plsc.load_gather(u_v, [dj]))
        m = s[0]
        for j in range(1, k):
            m = jnp.maximum(m, s[j])
        e = [jnp.exp(x - m) for x in s]
        tot = e[0]
        for j in range(1, k):
            tot = tot + e[j]
        best = e[0] / tot + noise_v[pl.ds(gbase, _L)]
        bestj = _full(0)
        for j in range(1, k):
            p = e[j] / tot + noise_v[pl.ds(gbase + j * _L, _L)]
            gt = p > best
            bestj = jnp.where(gt, _full(j), bestj)
            best = jnp.where(gt, p, best)
        chosen = plsc.load_gather(cand_v, [off + bestj])
        out_v[pl.ds(g * _L, _L)] = chosen


@functools.lru_cache(maxsize=4)
def _build_sc_router(n, npad, d, k):
    b = npad // _NW
    groups = b // _L
    bk = b * k
    tail = n - (_NW - 1) * b
    mesh = plsc.VectorSubcoreMesh(
        core_axis_name="c", subcore_axis_name="s",
        num_cores=_NC, num_subcores=_NS)

    @functools.partial(
        pl.kernel,
        out_type=[
            jax.ShapeDtypeStruct((n,), jnp.int32),
            jax.ShapeDtypeStruct((n,), jnp.int32),
            jax.ShapeDtypeStruct((n, 2 * d), jnp.float32),
        ],
        mesh=mesh,
        compiler_params=pltpu.CompilerParams(
            needs_layout_passes=False, disable_bounds_checks=True,
            disable_semaphore_checks=True),
        scratch_types=[
            pltpu.VMEM((n,), jnp.float32),
            pltpu.VMEM((n,), jnp.float32),
            pltpu.VMEM((bk,), jnp.int32),
            pltpu.VMEM((bk,), jnp.float32),
            pltpu.VMEM((bk,), jnp.int32),
            pltpu.VMEM((b,), jnp.int32),
            pltpu.VMEM((b,), jnp.int32),
            pltpu.VMEM((b, d), jnp.float32),
            pltpu.VMEM((b, d), jnp.float32),
            pltpu.SemaphoreType.DMA,
            pltpu.SemaphoreType.DMA,
            pltpu.SemaphoreType.DMA,
            pltpu.SemaphoreType.DMA,
            pltpu.SemaphoreType.DMA,
            pltpu.SemaphoreType.DMA,
        ],
    )
    def sc_router(u8_hbm, adjf_hbm, nz_hbm, v_hbm,
                  w1_hbm, w2_hbm, emb_hbm,
                  u0_v, u1_v, cand_v, noise_v, sidx_v, w1_v, w2_v,
                  rows0_v, rows1_v,
                  s_u0, s_u1, s_cand, s_noise, s_e0, s_e1):
        wid = lax.axis_index("s") * _NC + lax.axis_index("c")
        base = wid * b
        nk = n * k

        d_u0 = pltpu.async_copy(u8_hbm.at[0], u0_v, s_u0)
        d_u1 = pltpu.async_copy(u8_hbm.at[1], u1_v, s_u1)
        d_n = pltpu.async_copy(nz_hbm.at[pl.ds(base * k, bk)], noise_v,
                               s_noise)

        @pl.when(wid < _NW - 1)
        def _load_full():
            pltpu.sync_copy(adjf_hbm.at[pl.ds(nk + base * k, bk)], cand_v)

        @pl.when(wid == _NW - 1)
        def _load_tail():
            pltpu.sync_copy(adjf_hbm.at[pl.ds(nk + base * k, tail * k)],
                            cand_v.at[pl.ds(0, tail * k)])
            zeros16 = jnp.zeros((_L,), jnp.int32)

            @plsc.parallel_loop(tail * k, bk, _L)
            def _zfill(i):
                cand_v[pl.ds(i, _L)] = zeros16

        d_u0.wait()
        d_n.wait()

        _routing_step(u0_v, cand_v, noise_v, w1_v, groups, k)

        iota16 = lax.iota(jnp.int32, _L) * k

        @plsc.parallel_loop(0, groups, unroll=2)
        def sidx_body(c):
            wchunk = w1_v[pl.ds(c * _L, _L)] * k + nk
            pos = c * (k * _L) + iota16
            for j in range(k):
                plsc.store_scatter(sidx_v, [pos + _full(j)],
                                   wchunk + _full(j))

        c_ds = []
        for c in range(groups):
            for h in range(2):
                off = c * (k * _L) + h * 128
                c_ds.append(pltpu.async_copy(
                    adjf_hbm.at[sidx_v.at[pl.ds(off, 128)]],
                    cand_v.at[pl.ds(off, 128)], s_cand))
        d_n1 = pltpu.async_copy(
            nz_hbm.at[pl.ds(npad * k + base * k, bk)], noise_v, s_noise)
        e0_ds = []
        for c in range(groups):
            e0_ds.append(pltpu.async_copy(
                v_hbm.at[w1_v.at[pl.ds(c * _L, _L)]],
                rows0_v.at[pl.ds(c * _L, _L)], s_e0))
        for ds in c_ds:
            ds.wait()
        d_n1.wait()
        d_u1.wait()

        _routing_step(u1_v, cand_v, noise_v, w2_v, groups, k)

        e1_ds = []
        for c in range(groups):
            e1_ds.append(pltpu.async_copy(
                v_hbm.at[w2_v.at[pl.ds(c * _L, _L)]],
                rows1_v.at[pl.ds(c * _L, _L)], s_e1))
        for ds in e0_ds:
            ds.wait()
        for ds in e1_ds:
            ds.wait()

        @pl.when(wid < _NW - 1)
        def _full_write():
            pltpu.sync_copy(w1_v, w1_hbm.at[pl.ds(base, b)])
            pltpu.sync_copy(w2_v, w2_hbm.at[pl.ds(base, b)])
            pltpu.sync_copy(rows0_v, emb_hbm.at[pl.ds(base, b), pl.ds(0, d)])
            pltpu.sync_copy(rows1_v, emb_hbm.at[pl.ds(base, b), pl.ds(d, d)])

        @pl.when(wid == _NW - 1)
        def _tail_write():
            pltpu.sync_copy(w1_v.at[pl.ds(0, tail)],
                            w1_hbm.at[pl.ds(base, tail)])
            pltpu.sync_copy(w2_v.at[pl.ds(0, tail)],
                            w2_hbm.at[pl.ds(base, tail)])
            pltpu.sync_copy(rows0_v.at[pl.ds(0, tail)],
                            emb_hbm.at[pl.ds(base, tail), pl.ds(0, d)])
            pltpu.sync_copy(rows1_v.at[pl.ds(0, tail)],
                            emb_hbm.at[pl.ds(base, tail), pl.ds(d, d)])

    return sc_router


def _u_body(w_ref, v_ref, o_ref):
    o_ref[...] = lax.dot_general(
        w_ref[...], v_ref[...], (((1,), (1,)), ((), ())),
        preferred_element_type=jnp.float32)


def kernel(v, adj, slices, W, b):
    n, d = v.shape
    k = adj.shape[1] // slices.shape[0]
    t = W.shape[0] // d
    assert t == 2 and k == _L
    npad = -(-n // (_NW * _L)) * (_NW * _L)

    w_pad = jnp.zeros((8, d), jnp.float32).at[:t].set(W[:, 0].reshape(t, d))
    noise = []
    for i in range(t):
        ni = _EPS * jax.random.normal(
            jax.random.fold_in(jax.random.key(1234), i), (n * k,),
            dtype=jnp.float32)
        nip = jnp.zeros((npad, k), jnp.float32).at[:n].set(ni.reshape(n, k))
        noise.append(nip.reshape(-1, _L, k).transpose(0, 2, 1).reshape(-1))
    noise2 = jnp.concatenate(noise)

    u8 = pl.pallas_call(
        _u_body,
        out_shape=jax.ShapeDtypeStruct((8, n), jnp.float32),
    )(w_pad, v)

    w1, w2, walk_embeds = _build_sc_router(n, npad, d, k)(
        u8, adj.astype(jnp.int32).reshape(-1), noise2, v)

    walks = jnp.stack([jnp.arange(n, dtype=jnp.int32), w1, w2], axis=1)
    return walks, walk_embeds
